# Initial kernel scaffold; baseline (speedup 1.0000x reference)
import jax
import jax.numpy as jnp
from jax.experimental import pallas as pl


def kernel(x, edge_index, W0, b0, W1, b1):
    raise NotImplementedError("write your pallas kernel here")


# speedup vs baseline: 16.1339x; 16.1339x over previous
import functools

import jax
import jax.numpy as jnp
from jax import lax
from jax.experimental import pallas as pl
from jax.experimental.pallas import tpu as pltpu
from jax.experimental.pallas import tpu_sc as plsc

N = 10000
E = 320000
D = 128
NC = 2
NS = 16
NW = NC * NS
K = 128
EPAD = 327680
EPT = EPAD // NW
CHUNKS = EPT // K
NPAD = 10240
ZR = NPAD // NS

_mesh = plsc.VectorSubcoreMesh(core_axis_name="c", subcore_axis_name="s")


@functools.partial(
    pl.kernel,
    out_type=jax.ShapeDtypeStruct((NC, NPAD), jnp.float32),
    mesh=_mesh,
    scratch_types=[
        pltpu.VMEM_SHARED((NPAD,), jnp.float32),
        pltpu.VMEM((K,), jnp.int32),
        pltpu.VMEM((K,), jnp.float32),
        pltpu.VMEM((ZR,), jnp.float32),
    ],
)
def _hist_k(dst_hbm, out_hbm, hist_sh, dst_v, ones_v, zer_v):
    c = lax.axis_index("c")
    s = lax.axis_index("s")
    wid = s * NC + c
    for i in range(ZR // 16):
        zer_v[pl.ds(i * 16, 16)] = jnp.zeros((16,), jnp.float32)
    for i in range(K // 16):
        ones_v[pl.ds(i * 16, 16)] = jnp.ones((16,), jnp.float32)
    pltpu.sync_copy(zer_v, hist_sh.at[pl.ds(s * ZR, ZR)])
    plsc.subcore_barrier()
    base = wid * EPT

    def body(i, carry):
        pltpu.sync_copy(dst_hbm.at[pl.ds(base + i * K, K)], dst_v)
        pltpu.sync_copy(ones_v, hist_sh.at[dst_v], add=True)
        return carry

    lax.fori_loop(0, CHUNKS, body, 0)
    plsc.subcore_barrier()
    pltpu.sync_copy(hist_sh.at[pl.ds(s * ZR, ZR)], out_hbm.at[c, pl.ds(s * ZR, ZR)])


@functools.partial(
    pl.kernel,
    out_type=jax.ShapeDtypeStruct((NC, NPAD, D), jnp.float32),
    mesh=_mesh,
    scratch_types=[
        pltpu.VMEM_SHARED((NPAD, D), jnp.float32),
        pltpu.VMEM((K,), jnp.int32),
        pltpu.VMEM((K,), jnp.int32),
        pltpu.VMEM((K, D), jnp.float32),
        pltpu.VMEM((16, D), jnp.float32),
        pltpu.SemaphoreType.DMA,
    ],
)
def _scatter_k(g_hbm, src_hbm, dst_hbm, out_hbm, acc_sh, src_v, dst_v, rows_v,
               zer_v, sem):
    c = lax.axis_index("c")
    s = lax.axis_index("s")
    wid = s * NC + c
    for r in range(16):
        for l in range(D // 16):
            zer_v[r, pl.ds(l * 16, 16)] = jnp.zeros((16,), jnp.float32)

    def zero_body(i, carry):
        pltpu.sync_copy(zer_v, acc_sh.at[pl.ds(s * ZR + i * 16, 16), :])
        return carry

    lax.fori_loop(0, ZR // 16, zero_body, 0)
    plsc.subcore_barrier()
    base = wid * EPT

    def body(i, carry):
        off = base + i * K
        pltpu.sync_copy(src_hbm.at[pl.ds(off, K)], src_v)
        pltpu.sync_copy(dst_hbm.at[pl.ds(off, K)], dst_v)
        pltpu.async_copy(g_hbm.at[src_v], rows_v, sem).wait()
        pltpu.sync_copy(rows_v, acc_sh.at[dst_v], add=True)
        return carry

    lax.fori_loop(0, CHUNKS, body, 0)
    plsc.subcore_barrier()
    r0 = s * ZR
    pltpu.sync_copy(acc_sh.at[pl.ds(r0, ZR), :],
                    out_hbm.at[c, pl.ds(r0, ZR), :])


def _dinv_body(hp_ref, o_ref):
    o_ref[...] = lax.rsqrt(hp_ref[0] + hp_ref[1] + 1.0)


def _dinv(hist_p):
    return pl.pallas_call(
        _dinv_body,
        out_shape=jax.ShapeDtypeStruct((NPAD // D, D), jnp.float32),
    )(hist_p.reshape(2, NPAD // D, D))


_BR = 1000
_GRID = N // _BR


def _l0_body(x_ref, w_ref, dv_ref, o_ref):
    o_ref[...] = dv_ref[...] * jnp.dot(
        x_ref[...], w_ref[...], preferred_element_type=jnp.float32)


def _layer0(x, W0, dinv2):
    return pl.pallas_call(
        _l0_body,
        grid=(_GRID,),
        in_specs=[
            pl.BlockSpec((_BR, D), lambda i: (i, 0)),
            pl.BlockSpec((D, D), lambda i: (0, 0)),
            pl.BlockSpec((_BR, 1), lambda i: (i, 0)),
        ],
        out_specs=pl.BlockSpec((_BR, D), lambda i: (i, 0)),
        out_shape=jax.ShapeDtypeStruct((N, D), jnp.float32),
    )(x, W0, dinv2)


def _l1_body(pa_ref, pb_ref, g_ref, dv_ref, b_ref, w_ref, o_ref):
    dv = dv_ref[...]
    h = dv * (pa_ref[0] + pb_ref[0] + g_ref[...]) + b_ref[...]
    h = jnp.where(h > 0, h, 0.01 * h)
    o_ref[...] = dv * jnp.dot(h, w_ref[...], preferred_element_type=jnp.float32)


def _layer1(p, g0, dinv2, b0r, W1):
    return pl.pallas_call(
        _l1_body,
        grid=(_GRID,),
        in_specs=[
            pl.BlockSpec((1, _BR, D), lambda i: (0, i, 0)),
            pl.BlockSpec((1, _BR, D), lambda i: (1, i, 0)),
            pl.BlockSpec((_BR, D), lambda i: (i, 0)),
            pl.BlockSpec((_BR, 1), lambda i: (i, 0)),
            pl.BlockSpec((1, D), lambda i: (0, 0)),
            pl.BlockSpec((D, D), lambda i: (0, 0)),
        ],
        out_specs=pl.BlockSpec((_BR, D), lambda i: (i, 0)),
        out_shape=jax.ShapeDtypeStruct((N, D), jnp.float32),
    )(p, p, g0, dinv2, b0r, W1)


def _fin_body(pa_ref, pb_ref, g_ref, dv_ref, b_ref, o_ref):
    o_ref[...] = dv_ref[...] * (pa_ref[0] + pb_ref[0] + g_ref[...]) + b_ref[...]


def _final(p, g1, dinv2, b1r):
    return pl.pallas_call(
        _fin_body,
        grid=(_GRID,),
        in_specs=[
            pl.BlockSpec((1, _BR, D), lambda i: (0, i, 0)),
            pl.BlockSpec((1, _BR, D), lambda i: (1, i, 0)),
            pl.BlockSpec((_BR, D), lambda i: (i, 0)),
            pl.BlockSpec((_BR, 1), lambda i: (i, 0)),
            pl.BlockSpec((1, D), lambda i: (0, 0)),
        ],
        out_specs=pl.BlockSpec((_BR, D), lambda i: (i, 0)),
        out_shape=jax.ShapeDtypeStruct((N, D), jnp.float32),
    )(p, p, g1, dinv2, b1r)


def kernel(x, edge_index, W0, b0, W1, b1):
    src = edge_index[0].astype(jnp.int32)
    dst = edge_index[1].astype(jnp.int32)
    pad = EPAD - E
    ar = jnp.arange(pad, dtype=jnp.int32)
    srcp = jnp.concatenate([src, ar % N])
    dstp = jnp.concatenate([dst, N + ar % (NPAD - N)])

    hist_p = _hist_k(dstp)
    dinv2 = _dinv(hist_p).reshape(NPAD)[:N].reshape(N, 1)

    g0 = _layer0(x, W0, dinv2)
    p0 = _scatter_k(g0, srcp, dstp)
    g1 = _layer1(p0, g0, dinv2, b0.reshape(1, D), W1)
    p1 = _scatter_k(g1, srcp, dstp)
    return _final(p1, g1, dinv2, b1.reshape(1, D))


# speedup vs baseline: 30.4945x; 1.8901x over previous
import functools

import jax
import jax.numpy as jnp
from jax import lax
from jax.experimental import pallas as pl
from jax.experimental.pallas import tpu as pltpu
from jax.experimental.pallas import tpu_sc as plsc

N = 10000
E = 320000
D = 128
NC = 2
NS = 16
NW = NC * NS
K = 128
EPAD = 327680
EPT = EPAD // NW
CHUNKS = EPT // K
NPAD = 10240
ZR = NPAD // NS

_mesh = plsc.VectorSubcoreMesh(core_axis_name="c", subcore_axis_name="s")


_HGRP = 8


@functools.partial(
    pl.kernel,
    out_type=jax.ShapeDtypeStruct((NC, NPAD), jnp.float32),
    mesh=_mesh,
    scratch_types=[
        pltpu.VMEM_SHARED((NPAD,), jnp.float32),
        pltpu.VMEM((CHUNKS, K), jnp.int32),
        pltpu.VMEM((K,), jnp.float32),
        pltpu.VMEM((ZR,), jnp.float32),
        pltpu.SemaphoreType.DMA,
    ],
)
def _hist_k(dst_hbm, out_hbm, hist_sh, dst_all, ones_v, zer_v, sem):
    c = lax.axis_index("c")
    s = lax.axis_index("s")
    wid = s * NC + c
    for i in range(ZR // 16):
        zer_v[pl.ds(i * 16, 16)] = jnp.zeros((16,), jnp.float32)
    for i in range(K // 16):
        ones_v[pl.ds(i * 16, 16)] = jnp.ones((16,), jnp.float32)
    pltpu.sync_copy(dst_hbm.at[wid], dst_all)
    pltpu.sync_copy(zer_v, hist_sh.at[pl.ds(s * ZR, ZR)])
    plsc.subcore_barrier()

    def body(k, carry):
        descs = []
        for b in range(_HGRP):
            descs.append(pltpu.async_copy(
                ones_v, hist_sh.at[dst_all.at[k * _HGRP + b]], sem, add=True))
        for d in descs:
            d.wait()
        return carry

    lax.fori_loop(0, CHUNKS // _HGRP, body, 0)
    plsc.subcore_barrier()
    pltpu.sync_copy(hist_sh.at[pl.ds(s * ZR, ZR)], out_hbm.at[c, pl.ds(s * ZR, ZR)])


_DEPTH = 2


@functools.partial(
    pl.kernel,
    out_type=jax.ShapeDtypeStruct((NC, NPAD, D), jnp.float32),
    mesh=_mesh,
    scratch_types=[
        pltpu.VMEM_SHARED((NPAD, D), jnp.float32),
        pltpu.VMEM((_DEPTH, K), jnp.int32),
        pltpu.VMEM((_DEPTH, K), jnp.int32),
        pltpu.VMEM((_DEPTH, K, D), jnp.float32),
        pltpu.VMEM((16, D), jnp.float32),
        [pltpu.SemaphoreType.DMA] * _DEPTH,
        [pltpu.SemaphoreType.DMA] * _DEPTH,
    ],
)
def _scatter_k(g_hbm, src_hbm, dst_hbm, out_hbm, acc_sh, src_v, dst_v,
               rows_v, zer_v, gsems, isems):
    c = lax.axis_index("c")
    s = lax.axis_index("s")
    wid = s * NC + c
    for r in range(16):
        for l in range(D // 16):
            zer_v[r, pl.ds(l * 16, 16)] = jnp.zeros((16,), jnp.float32)

    def zero_body(i, carry):
        pltpu.sync_copy(zer_v, acc_sh.at[pl.ds(s * ZR + i * 16, 16), :])
        return carry

    lax.fori_loop(0, ZR // 16, zero_body, 0)

    def idx_load(b, i):
        pltpu.async_copy(src_hbm.at[wid, i], src_v.at[b], isems[b])
        pltpu.async_copy(dst_hbm.at[wid, i], dst_v.at[b], isems[b])

    def idx_drain(b):
        pltpu.make_async_copy(src_hbm.at[wid, 0], src_v.at[b], isems[b]).wait()
        pltpu.make_async_copy(dst_hbm.at[wid, 0], dst_v.at[b], isems[b]).wait()

    def gather(b):
        pltpu.async_copy(g_hbm.at[src_v.at[b]], rows_v.at[b], gsems[b])

    def gather_drain(b):
        pltpu.make_async_copy(g_hbm.at[src_v.at[0]], rows_v.at[b],
                              gsems[b]).wait()

    plsc.subcore_barrier()
    for b in range(_DEPTH):
        idx_load(b, b)
    idx_drain(0)
    gather(0)

    def body(k, carry):
        for b in range(_DEPTH):
            i = k * _DEPTH + b
            nb = (b + 1) % _DEPTH

            @pl.when(i + 1 < CHUNKS)
            def _():
                idx_drain(nb)
                gather(nb)

            gather_drain(b)
            pltpu.sync_copy(rows_v.at[b], acc_sh.at[dst_v.at[b]], add=True)

            @pl.when(i + _DEPTH < CHUNKS)
            def _():
                idx_load(b, i + _DEPTH)
        return carry

    lax.fori_loop(0, CHUNKS // _DEPTH, body, 0)
    plsc.subcore_barrier()
    r0 = s * ZR
    pltpu.sync_copy(acc_sh.at[pl.ds(r0, ZR), :],
                    out_hbm.at[c, pl.ds(r0, ZR), :])


def _dinv_body(hp_ref, o_ref):
    o_ref[...] = lax.rsqrt(hp_ref[0] + hp_ref[1] + 1.0)


def _dinv(hist_p):
    return pl.pallas_call(
        _dinv_body,
        out_shape=jax.ShapeDtypeStruct((NPAD // D, D), jnp.float32),
    )(hist_p.reshape(2, NPAD // D, D))


_BR = 1000
_GRID = N // _BR


def _l0_body(x_ref, w_ref, dv_ref, o_ref):
    o_ref[...] = dv_ref[...] * jnp.dot(
        x_ref[...], w_ref[...], preferred_element_type=jnp.float32)


def _layer0(x, W0, dinv2):
    return pl.pallas_call(
        _l0_body,
        grid=(_GRID,),
        in_specs=[
            pl.BlockSpec((_BR, D), lambda i: (i, 0)),
            pl.BlockSpec((D, D), lambda i: (0, 0)),
            pl.BlockSpec((_BR, 1), lambda i: (i, 0)),
        ],
        out_specs=pl.BlockSpec((_BR, D), lambda i: (i, 0)),
        out_shape=jax.ShapeDtypeStruct((N, D), jnp.float32),
    )(x, W0, dinv2)


def _l1_body(pa_ref, pb_ref, g_ref, dv_ref, b_ref, w_ref, o_ref):
    dv = dv_ref[...]
    h = dv * (pa_ref[0] + pb_ref[0] + g_ref[...]) + b_ref[...]
    h = jnp.where(h > 0, h, 0.01 * h)
    o_ref[...] = dv * jnp.dot(h, w_ref[...], preferred_element_type=jnp.float32)


def _layer1(p, g0, dinv2, b0r, W1):
    return pl.pallas_call(
        _l1_body,
        grid=(_GRID,),
        in_specs=[
            pl.BlockSpec((1, _BR, D), lambda i: (0, i, 0)),
            pl.BlockSpec((1, _BR, D), lambda i: (1, i, 0)),
            pl.BlockSpec((_BR, D), lambda i: (i, 0)),
            pl.BlockSpec((_BR, 1), lambda i: (i, 0)),
            pl.BlockSpec((1, D), lambda i: (0, 0)),
            pl.BlockSpec((D, D), lambda i: (0, 0)),
        ],
        out_specs=pl.BlockSpec((_BR, D), lambda i: (i, 0)),
        out_shape=jax.ShapeDtypeStruct((N, D), jnp.float32),
    )(p, p, g0, dinv2, b0r, W1)


def _fin_body(pa_ref, pb_ref, g_ref, dv_ref, b_ref, o_ref):
    o_ref[...] = dv_ref[...] * (pa_ref[0] + pb_ref[0] + g_ref[...]) + b_ref[...]


def _final(p, g1, dinv2, b1r):
    return pl.pallas_call(
        _fin_body,
        grid=(_GRID,),
        in_specs=[
            pl.BlockSpec((1, _BR, D), lambda i: (0, i, 0)),
            pl.BlockSpec((1, _BR, D), lambda i: (1, i, 0)),
            pl.BlockSpec((_BR, D), lambda i: (i, 0)),
            pl.BlockSpec((_BR, 1), lambda i: (i, 0)),
            pl.BlockSpec((1, D), lambda i: (0, 0)),
        ],
        out_specs=pl.BlockSpec((_BR, D), lambda i: (i, 0)),
        out_shape=jax.ShapeDtypeStruct((N, D), jnp.float32),
    )(p, p, g1, dinv2, b1r)


def kernel(x, edge_index, W0, b0, W1, b1):
    src = edge_index[0].astype(jnp.int32)
    dst = edge_index[1].astype(jnp.int32)
    pad = EPAD - E
    ar = jnp.arange(pad, dtype=jnp.int32)
    srcp = jnp.concatenate([src, ar % N]).reshape(NW, CHUNKS, K)
    dstp = jnp.concatenate([dst, N + ar % (NPAD - N)]).reshape(NW, CHUNKS, K)

    hist_p = _hist_k(dstp)
    dinv2 = _dinv(hist_p).reshape(NPAD)[:N].reshape(N, 1)

    g0 = _layer0(x, W0, dinv2)
    p0 = _scatter_k(g0, srcp, dstp)
    g1 = _layer1(p0, g0, dinv2, b0.reshape(1, D), W1)
    p1 = _scatter_k(g1, srcp, dstp)
    return _final(p1, g1, dinv2, b1.reshape(1, D))


# speedup vs baseline: 33.7383x; 1.1064x over previous
import functools

import jax
import jax.numpy as jnp
from jax import lax
from jax.experimental import pallas as pl
from jax.experimental.pallas import tpu as pltpu
from jax.experimental.pallas import tpu_sc as plsc

N = 10000
E = 320000
D = 128
NC = 2
NS = 16
NW = NC * NS
K = 128
EPAD = 327680
EPT = EPAD // NW
CHUNKS = EPT // K
NPAD = 10240
ZR = NPAD // NS

_mesh = plsc.VectorSubcoreMesh(core_axis_name="c", subcore_axis_name="s")


_HGRP = 8


@functools.partial(
    pl.kernel,
    out_type=jax.ShapeDtypeStruct((NC, NPAD), jnp.float32),
    mesh=_mesh,
    scratch_types=[
        pltpu.VMEM_SHARED((NPAD,), jnp.float32),
        pltpu.VMEM((CHUNKS, K), jnp.int32),
        pltpu.VMEM((K,), jnp.float32),
        pltpu.VMEM((ZR,), jnp.float32),
        pltpu.SemaphoreType.DMA,
    ],
)
def _hist_k(dst_hbm, out_hbm, hist_sh, dst_all, ones_v, zer_v, sem):
    c = lax.axis_index("c")
    s = lax.axis_index("s")
    wid = s * NC + c
    for i in range(ZR // 16):
        zer_v[pl.ds(i * 16, 16)] = jnp.zeros((16,), jnp.float32)
    for i in range(K // 16):
        ones_v[pl.ds(i * 16, 16)] = jnp.ones((16,), jnp.float32)
    pltpu.sync_copy(dst_hbm.at[wid], dst_all)
    pltpu.sync_copy(zer_v, hist_sh.at[pl.ds(s * ZR, ZR)])
    plsc.subcore_barrier()

    def body(k, carry):
        descs = []
        for b in range(_HGRP):
            descs.append(pltpu.async_copy(
                ones_v, hist_sh.at[dst_all.at[k * _HGRP + b]], sem, add=True))
        for d in descs:
            d.wait()
        return carry

    lax.fori_loop(0, CHUNKS // _HGRP, body, 0)
    plsc.subcore_barrier()
    pltpu.sync_copy(hist_sh.at[pl.ds(s * ZR, ZR)], out_hbm.at[c, pl.ds(s * ZR, ZR)])


_DEPTH = 2


@functools.partial(
    pl.kernel,
    out_type=jax.ShapeDtypeStruct((NC, NPAD, D), jnp.float32),
    mesh=_mesh,
    scratch_types=[
        pltpu.VMEM_SHARED((NPAD, D), jnp.float32),
        pltpu.VMEM((_DEPTH, K), jnp.int32),
        pltpu.VMEM((_DEPTH, K), jnp.int32),
        pltpu.VMEM((_DEPTH, K, D), jnp.float32),
        pltpu.VMEM((16, D), jnp.float32),
        [pltpu.SemaphoreType.DMA] * _DEPTH,
        [pltpu.SemaphoreType.DMA] * _DEPTH,
        [pltpu.SemaphoreType.DMA] * _DEPTH,
    ],
)
def _scatter_k(g_hbm, src_hbm, dst_hbm, out_hbm, acc_sh, src_v, dst_v,
               rows_v, zer_v, gsems, isems, ssems):
    c = lax.axis_index("c")
    s = lax.axis_index("s")
    wid = s * NC + c
    for r in range(16):
        for l in range(D // 16):
            zer_v[r, pl.ds(l * 16, 16)] = jnp.zeros((16,), jnp.float32)

    def zero_body(i, carry):
        pltpu.sync_copy(zer_v, acc_sh.at[pl.ds(s * ZR + i * 16, 16), :])
        return carry

    lax.fori_loop(0, ZR // 16, zero_body, 0)

    def idx_load(b, i):
        pltpu.async_copy(src_hbm.at[wid, i], src_v.at[b], isems[b])
        pltpu.async_copy(dst_hbm.at[wid, i], dst_v.at[b], isems[b])

    def idx_drain(b):
        pltpu.make_async_copy(src_hbm.at[wid, 0], src_v.at[b], isems[b]).wait()
        pltpu.make_async_copy(dst_hbm.at[wid, 0], dst_v.at[b], isems[b]).wait()

    def gather(b):
        pltpu.async_copy(g_hbm.at[src_v.at[b]], rows_v.at[b], gsems[b])

    def gather_drain(b):
        pltpu.make_async_copy(g_hbm.at[src_v.at[0]], rows_v.at[b],
                              gsems[b]).wait()

    def scatter(b):
        pltpu.async_copy(rows_v.at[b], acc_sh.at[dst_v.at[b]], ssems[b],
                         add=True)

    def scatter_drain(b):
        pltpu.make_async_copy(rows_v.at[b], acc_sh.at[dst_v.at[0]],
                              ssems[b]).wait()

    plsc.subcore_barrier()
    for b in range(_DEPTH):
        idx_load(b, b)
    idx_drain(0)
    gather(0)

    def body(k, carry):
        for b in range(_DEPTH):
            i = k * _DEPTH + b
            nb = (b + 1) % _DEPTH

            @pl.when(i + 1 < CHUNKS)
            def _():
                idx_drain(nb)

                @pl.when(i >= 1)
                def _():
                    scatter_drain(nb)

                gather(nb)

            gather_drain(b)
            scatter(b)

            @pl.when(i + _DEPTH < CHUNKS)
            def _():
                idx_load(b, i + _DEPTH)
        return carry

    lax.fori_loop(0, CHUNKS // _DEPTH, body, 0)
    for b in range(_DEPTH):
        scatter_drain(b)
    plsc.subcore_barrier()
    r0 = s * ZR
    pltpu.sync_copy(acc_sh.at[pl.ds(r0, ZR), :],
                    out_hbm.at[c, pl.ds(r0, ZR), :])


def _dinv_body(hp_ref, o_ref):
    o_ref[...] = lax.rsqrt(hp_ref[0] + hp_ref[1] + 1.0)


def _dinv(hist_p):
    return pl.pallas_call(
        _dinv_body,
        out_shape=jax.ShapeDtypeStruct((NPAD // D, D), jnp.float32),
    )(hist_p.reshape(2, NPAD // D, D))


_BR = 1000
_GRID = N // _BR


def _l0_body(x_ref, w_ref, dv_ref, o_ref):
    o_ref[...] = dv_ref[...] * jnp.dot(
        x_ref[...], w_ref[...], preferred_element_type=jnp.float32)


def _layer0(x, W0, dinv2):
    return pl.pallas_call(
        _l0_body,
        grid=(_GRID,),
        in_specs=[
            pl.BlockSpec((_BR, D), lambda i: (i, 0)),
            pl.BlockSpec((D, D), lambda i: (0, 0)),
            pl.BlockSpec((_BR, 1), lambda i: (i, 0)),
        ],
        out_specs=pl.BlockSpec((_BR, D), lambda i: (i, 0)),
        out_shape=jax.ShapeDtypeStruct((N, D), jnp.float32),
    )(x, W0, dinv2)


def _l1_body(pa_ref, pb_ref, g_ref, dv_ref, b_ref, w_ref, o_ref):
    dv = dv_ref[...]
    h = dv * (pa_ref[0] + pb_ref[0] + g_ref[...]) + b_ref[...]
    h = jnp.where(h > 0, h, 0.01 * h)
    o_ref[...] = dv * jnp.dot(h, w_ref[...], preferred_element_type=jnp.float32)


def _layer1(p, g0, dinv2, b0r, W1):
    return pl.pallas_call(
        _l1_body,
        grid=(_GRID,),
        in_specs=[
            pl.BlockSpec((1, _BR, D), lambda i: (0, i, 0)),
            pl.BlockSpec((1, _BR, D), lambda i: (1, i, 0)),
            pl.BlockSpec((_BR, D), lambda i: (i, 0)),
            pl.BlockSpec((_BR, 1), lambda i: (i, 0)),
            pl.BlockSpec((1, D), lambda i: (0, 0)),
            pl.BlockSpec((D, D), lambda i: (0, 0)),
        ],
        out_specs=pl.BlockSpec((_BR, D), lambda i: (i, 0)),
        out_shape=jax.ShapeDtypeStruct((N, D), jnp.float32),
    )(p, p, g0, dinv2, b0r, W1)


def _fin_body(pa_ref, pb_ref, g_ref, dv_ref, b_ref, o_ref):
    o_ref[...] = dv_ref[...] * (pa_ref[0] + pb_ref[0] + g_ref[...]) + b_ref[...]


def _final(p, g1, dinv2, b1r):
    return pl.pallas_call(
        _fin_body,
        grid=(_GRID,),
        in_specs=[
            pl.BlockSpec((1, _BR, D), lambda i: (0, i, 0)),
            pl.BlockSpec((1, _BR, D), lambda i: (1, i, 0)),
            pl.BlockSpec((_BR, D), lambda i: (i, 0)),
            pl.BlockSpec((_BR, 1), lambda i: (i, 0)),
            pl.BlockSpec((1, D), lambda i: (0, 0)),
        ],
        out_specs=pl.BlockSpec((_BR, D), lambda i: (i, 0)),
        out_shape=jax.ShapeDtypeStruct((N, D), jnp.float32),
    )(p, p, g1, dinv2, b1r)


def kernel(x, edge_index, W0, b0, W1, b1):
    src = edge_index[0].astype(jnp.int32)
    dst = edge_index[1].astype(jnp.int32)
    pad = EPAD - E
    ar = jnp.arange(pad, dtype=jnp.int32)
    srcp = jnp.concatenate([src, ar % N]).reshape(NW, CHUNKS, K)
    dstp = jnp.concatenate([dst, N + ar % (NPAD - N)]).reshape(NW, CHUNKS, K)

    hist_p = _hist_k(dstp)
    dinv2 = _dinv(hist_p).reshape(NPAD)[:N].reshape(N, 1)

    g0 = _layer0(x, W0, dinv2)
    p0 = _scatter_k(g0, srcp, dstp)
    g1 = _layer1(p0, g0, dinv2, b0.reshape(1, D), W1)
    p1 = _scatter_k(g1, srcp, dstp)
    return _final(p1, g1, dinv2, b1.reshape(1, D))
